# Initial kernel scaffold; baseline (speedup 1.0000x reference)
import jax
import jax.numpy as jnp
from jax.experimental import pallas as pl


def kernel(attns_maps, pos_inds):
    raise NotImplementedError("write your pallas kernel here")


# speedup vs baseline: 299.0603x; 299.0603x over previous
import functools

import jax
import jax.numpy as jnp
from jax.experimental import pallas as pl
from jax.experimental.pallas import tpu as pltpu

_BLOCKS = 12
_BS = 8
_N = 677
_NUM_POINTS = 100
_NUM_GT = 4
_SCALE = 4
_NUM_PATCHES = _N - 1 - _NUM_POINTS
_KK = int(_N * _N * 0.5)
_BISECT_ITERS = 22


def _rollout_kernel(pos_ref, attn_ref, out_ref, w_ref):
    b = pl.program_id(0)
    j = pl.program_id(1)

    a = attn_ref[0, 0]

    def bisect(_, carry):
        lo, hi = carry
        mid = 0.5 * (lo + hi)
        cnt = jnp.sum((a <= mid).astype(jnp.float32))
        pred = cnt >= float(_KK)
        return jnp.where(pred, lo, mid), jnp.where(pred, mid, hi)

    lo, _ = jax.lax.fori_loop(0, _BISECT_ITERS, bisect, (0.0, 1.0))
    f = jnp.where(a > lo, a, 0.0)

    inv = 1.0 / (jnp.sum(f, axis=1) + 1.0)

    @pl.when(j == 0)
    def _start():
        for g in range(_NUM_GT):
            r = _N - _NUM_POINTS + pos_ref[b, g]
            raw = attn_ref[0, 0, pl.ds(r, 1), :]
            row = jnp.where(raw > lo, raw, 0.0)
            col = jax.lax.broadcasted_iota(jnp.int32, (1, _N), 1)
            row = row + jnp.where(col == r, 1.0, 0.0)
            row = row * (1.0 / jnp.sum(row))
            w_ref[g, :] = row[0, :]
            out_ref[0, 0, g, :] = row[0, 1 : 1 + _NUM_PATCHES]

    @pl.when(j > 0)
    def _step():
        w = w_ref[0:_NUM_GT, :]
        u = w * inv[None, :]
        w_new = jnp.dot(u, f, preferred_element_type=jnp.float32) + u
        w_ref[0:_NUM_GT, :] = w_new
        out_ref[0, 0, :, :] = w_new[:, 1 : 1 + _NUM_PATCHES]


def kernel(attns_maps, pos_inds):
    pos = pos_inds.astype(jnp.int32)

    grid_spec = pltpu.PrefetchScalarGridSpec(
        num_scalar_prefetch=1,
        grid=(_BS, _SCALE),
        in_specs=[
            pl.BlockSpec(
                (1, 1, _N, _N),
                lambda b, j, pos_ref: (_BLOCKS - 1 - j, b, 0, 0),
            ),
        ],
        out_specs=pl.BlockSpec(
            (1, 1, _NUM_GT, _NUM_PATCHES),
            lambda b, j, pos_ref: (b, _SCALE - 1 - j, 0, 0),
        ),
        scratch_shapes=[pltpu.VMEM((8, _N), jnp.float32)],
    )

    out = pl.pallas_call(
        _rollout_kernel,
        grid_spec=grid_spec,
        out_shape=jax.ShapeDtypeStruct(
            (_BS, _SCALE, _NUM_GT, _NUM_PATCHES), jnp.float32
        ),
    )(pos, attns_maps)

    return jnp.transpose(out, (0, 2, 1, 3)).reshape(
        _BS * _NUM_GT, _SCALE, _NUM_PATCHES
    )
